# Initial kernel scaffold; baseline (speedup 1.0000x reference)
import jax
import jax.numpy as jnp
from jax.experimental import pallas as pl


def kernel(x, mask, x_interpolate, params):
    raise NotImplementedError("write your pallas kernel here")


# speedup vs baseline: 281.8243x; 281.8243x over previous
import functools

import jax
import jax.numpy as jnp
import numpy as np
from jax import lax
from jax.experimental import pallas as pl
from jax.experimental.pallas import tpu as pltpu
from jax.experimental.pallas import tpu_sc as plsc

B0, L0, C0 = 64, 1024, 64
K0, R0 = 3, 3
EPS = 1e-5
N2 = L0 // 2
P_MAX = L0 // 2
PAD = R0 * P_MAX
LPAD = L0 + 2 * PAD
NW = 32
NG = (B0 * C0) // 16
GPW = NG // NW

_GW = [float(np.exp(-1.0 * i * i / 2.0) / np.sqrt(2.0 * np.pi))
       for i in range(1, R0 + 1)]


def _l2norm_cols(v):
    n = jnp.sqrt(jnp.sum(v * v, axis=0, keepdims=True))
    return v / jnp.maximum(n, 1e-12)


def _roll_lanes(v, shift):
    if shift == 1:
        return jnp.concatenate([v[:, -1:], v[:, :-1]], axis=1)
    return jnp.concatenate([v[:, 1:], v[:, :1]], axis=1)


def _conv_over_l(v, w8):
    t0 = jnp.sum(v * w8[:, 0:1], axis=0, keepdims=True)
    t1 = jnp.sum(v * w8[:, 1:2], axis=0, keepdims=True)
    t2 = jnp.sum(v * w8[:, 2:3], axis=0, keepdims=True)
    return _roll_lanes(t0, 1) + t1 + _roll_lanes(t2, -1)


def _mlp23(h, w1t, b1, w2t, b2, w3t):
    h = jax.nn.relu(jnp.dot(h, w1t, preferred_element_type=jnp.float32) + b1)
    h = jax.nn.relu(jnp.dot(h, w2t, preferred_element_type=jnp.float32) + b2)
    return jnp.dot(h, w3t, preferred_element_type=jnp.float32)


def _k1_body(x_ref, xi_ref, cos_ref, sin_ref,
             cvt_ref, w1t_t, b1_t, w2t_t, b2_t, w3t_t,
             cvp_ref, w1t_p, b1_p, w2t_p, b2_p, w3t_p,
             cvx_ref,
             p_out, a_out, misc_out):
    x = x_ref[0]
    xi = xi_ref[0]
    xd = _l2norm_cols(x)
    xint = _l2norm_cols(xi)

    re = jnp.dot(cos_ref[...], xi, preferred_element_type=jnp.float32)
    im = jnp.dot(sin_ref[...], xi, preferred_element_type=jnp.float32)
    amp = jnp.sqrt(re * re + im * im)

    iota = lax.broadcasted_iota(jnp.int32, (N2, C0), 0)

    def pick_max(a):
        m = jnp.max(a, axis=0, keepdims=True)
        idx = jnp.min(jnp.where(a == m, iota, N2), axis=0, keepdims=True)
        a2 = jnp.where(iota == idx, -1.0, a)
        return m, idx, a2

    m1, i1, amp_r = pick_max(amp)
    m2, i2, amp_r = pick_max(amp_r)
    m3, i3, _ = pick_max(amp_r)
    amp_sum = m1 + m2 + m3 + EPS

    def period_of(idx):
        jf = idx.astype(jnp.float32)
        return jnp.where(idx <= 1, 1.0, jnp.round(1024.0 / jf))

    mps = [period_of(i1), period_of(i2), period_of(i3)]
    rates = [m1 / amp_sum, m2 / amp_sum, m3 / amp_sum]

    s_trend = _conv_over_l(xd, cvt_ref[...])
    s_per = _conv_over_l(xint, cvp_ref[...])
    sx = _conv_over_l(xint, cvx_ref[...])

    ones_c = jnp.ones((1, C0), jnp.float32)
    multi_trend = jnp.exp(jnp.tanh(_mlp23(
        jnp.concatenate([s_trend, ones_c], axis=1),
        w1t_t[...], b1_t[...], w2t_t[...], b2_t[...], w3t_t[...])))

    a_ks = []
    for k in range(K0):
        mk = jnp.exp(jnp.tanh(_mlp23(
            jnp.concatenate([s_per, mps[k]], axis=1),
            w1t_p[...], b1_p[...], w2t_p[...], b2_p[...], w3t_p[...])))
        a_ks.append(rates[k] * mk)

    r0 = rates[0] + rates[1] + rates[2]

    p_out[0] = jnp.concatenate(mps, axis=1)
    a_out[0] = jnp.concatenate(a_ks, axis=1)
    misc_out[0] = jnp.concatenate([r0, multi_trend, sx], axis=1)


def _sc_weight_body(z_hbm, p_hbm, a_hbm, r0_hbm, out_hbm,
                    zbuf, obuf, ibuf, fbuf):
    cid = lax.axis_index("c")
    sid = lax.axis_index("s")
    wid = sid * 2 + cid
    lane = lax.broadcasted_iota(jnp.int32, (16,), 0)

    def do_group(g, carry):
        pltpu.sync_copy(z_hbm.at[g], zbuf)
        for k in range(K0):
            pltpu.sync_copy(p_hbm.at[k, g], ibuf.at[k])
            pltpu.sync_copy(a_hbm.at[k, g], fbuf.at[k])
        pltpu.sync_copy(r0_hbm.at[g], fbuf.at[K0])
        pks = [ibuf[k] for k in range(K0)]
        aks = [fbuf[k] for k in range(K0)]
        r0v = fbuf[K0]
        offs = [[i * pks[k] for i in range(1, R0 + 1)] for k in range(K0)]

        def do_s(s, c2):
            col = jnp.full((16,), s + PAD, jnp.int32)
            acc = r0v
            for k in range(K0):
                gs = jnp.zeros((16,), jnp.float32)
                for i in range(1, R0 + 1):
                    off = offs[k][i - 1]
                    gs = gs + _GW[i - 1] * (
                        plsc.load_gather(zbuf, [lane, col - off])
                        + plsc.load_gather(zbuf, [lane, col + off]))
                acc = acc + aks[k] * gs
            plsc.store_scatter(obuf, [jnp.full((16,), s, jnp.int32), lane], acc)
            return c2

        lax.fori_loop(0, L0, do_s, 0)
        pltpu.sync_copy(obuf, out_hbm.at[g])
        return carry

    lax.fori_loop(wid * GPW, (wid + 1) * GPW, do_group, 0)


def _sc_weight(ztp, p_arr, a_arr, r0_arr, interpret=False):
    mesh = plsc.VectorSubcoreMesh(core_axis_name="c", subcore_axis_name="s")
    fn = pl.kernel(
        _sc_weight_body,
        out_type=jax.ShapeDtypeStruct((NG, L0, 16), jnp.float32),
        mesh=mesh,
        scratch_types=[
            pltpu.VMEM((16, LPAD), jnp.float32),
            pltpu.VMEM((L0, 16), jnp.float32),
            pltpu.VMEM((K0, 16), jnp.int32),
            pltpu.VMEM((K0 + 1, 16), jnp.float32),
        ],
        compiler_params=pltpu.CompilerParams(use_tc_tiling_on_sc=False, needs_layout_passes=False),
        interpret=interpret,
    )
    return fn(ztp, p_arr, a_arr, r0_arr)


def _k3_body(x_ref, m_ref, pw_ref, misc_ref,
             cvt_ref, cvp_ref, w1t_r, b1_r, w2t_r, b2_r, w3t_r,
             xn_out, w_out):
    x = x_ref[0]
    maskv = m_ref[0]
    pw = pw_ref[0]
    misc = misc_ref[0, 0]
    multi_trend = misc[C0:2 * C0][None]
    sx = misc[2 * C0:3 * C0][None]

    z = 1.0 - maskv
    gsum = jnp.zeros_like(z)
    zrow = jnp.zeros((1, C0), jnp.float32)
    for i in range(1, R0 + 1):
        zpadi = jnp.concatenate([zrow] * i, axis=0)
        down = jnp.concatenate([zpadi, z[:L0 - i]], axis=0)
        up = jnp.concatenate([z[i:], zpadi], axis=0)
        gsum = gsum + _GW[i - 1] * (down + up)
    tw = 1.0 + multi_trend * gsum

    s_t = _conv_over_l(tw, cvt_ref[...])
    s_p = _conv_over_l(pw, cvp_ref[...])
    h = jnp.concatenate([sx, s_t, s_p], axis=1)
    logits = _mlp23(h, w1t_r[...], b1_r[...], w2t_r[...], b2_r[...], w3t_r[...])
    ratio = 1.0 / (1.0 + jnp.exp(-logits))

    w = tw * ratio + pw * (1.0 - ratio)
    w = w * maskv
    xw = x * w
    cnt = jnp.sum(w, axis=0, keepdims=True)
    cnt = jnp.where(cnt == 0.0, 1.0, cnt)
    mean = jnp.sum(xw, axis=0, keepdims=True) / cnt
    xc = x - mean
    xc = jnp.where(maskv == 0.0, 0.0, xc)
    xw2 = xc * w
    stdev = jnp.sqrt(jnp.sum(xw2 * xw2, axis=0, keepdims=True) / cnt + EPS)
    xn_out[0] = xc / stdev
    w_out[0] = w


def _pad38(conv):
    return jnp.pad(conv, ((0, 0), (0, 5)))


def _row(v):
    return v[None].astype(jnp.float32)


@jax.jit
def kernel(x, mask, x_interpolate, params):
    jt = np.arange(N2, dtype=np.float64)[:, None]
    tt = np.arange(L0, dtype=np.float64)[None, :]
    ang = 2.0 * np.pi * jt * tt / L0
    cos_m = jnp.asarray(np.cos(ang), jnp.float32)
    sin_m = jnp.asarray(np.sin(ang), jnp.float32)

    ptm = params['trend_multi']
    ppm = params['period_multi']
    pr = params['ratio']

    full = lambda s: pl.BlockSpec(s, lambda b: (0,) * len(s))
    bspec = pl.BlockSpec((1, L0, C0), lambda b: (b, 0, 0))
    out_small = pl.BlockSpec((1, 1, 3 * C0), lambda b: (b, 0, 0))

    k1_in_specs = [bspec, bspec, full((N2, L0)), full((N2, L0))]
    k1_args = [x, x_interpolate, cos_m, sin_m]
    for p in (ptm, ppm):
        k1_args += [_pad38(p['conv']), p['w1'].T, _row(p['b1']),
                    p['w2'].T, _row(p['b2']), p['w3'].T]
        k1_in_specs += [full((L0, 8)), full((2 * C0, 16)), full((1, 16)),
                        full((16, 16)), full((1, 16)), full((16, C0))]
    k1_args.append(_pad38(pr['conv_x']))
    k1_in_specs.append(full((L0, 8)))

    p_pack, a_pack, misc = pl.pallas_call(
        _k1_body,
        grid=(B0,),
        in_specs=k1_in_specs,
        out_specs=[out_small, out_small, out_small],
        out_shape=[jax.ShapeDtypeStruct((B0, 1, 3 * C0), jnp.float32)] * 3,
    )(*k1_args)

    mp = p_pack.reshape(B0, K0, C0)
    p_arr = mp.astype(jnp.int32).transpose(1, 0, 2).reshape(K0, NG, 16)
    a_arr = a_pack.reshape(B0, K0, C0).transpose(1, 0, 2).reshape(K0, NG, 16)
    r0_arr = misc[:, 0, :C0].reshape(NG, 16)
    z = 1.0 - mask
    ztp = jnp.pad(jnp.swapaxes(z, 1, 2).reshape(B0 * C0, L0),
                  ((0, 0), (PAD, PAD))).reshape(NG, 16, LPAD)

    pw_g = _sc_weight(ztp, p_arr, a_arr, r0_arr)
    pw = pw_g.transpose(0, 2, 1).reshape(B0, C0, L0).swapaxes(1, 2)

    k3_in_specs = [bspec, bspec, bspec, out_small,
                   full((L0, 8)), full((L0, 8)), full((3 * C0, 16)),
                   full((1, 16)), full((16, 16)), full((1, 16)),
                   full((16, C0))]
    xn, w = pl.pallas_call(
        _k3_body,
        grid=(B0,),
        in_specs=k3_in_specs,
        out_specs=[bspec, bspec],
        out_shape=[jax.ShapeDtypeStruct((B0, L0, C0), jnp.float32)] * 2,
    )(x, mask, pw, misc,
      _pad38(pr['conv_t']), _pad38(pr['conv_p']), pr['w1'].T, _row(pr['b1']),
      pr['w2'].T, _row(pr['b2']), pr['w3'].T)
    return xn, w
